# baseline (speedup 1.0000x reference)
import jax
import jax.numpy as jnp
from jax import lax
from jax.experimental import pallas as pl
from jax.experimental.pallas import tpu as pltpu
from jax.experimental.pallas import tpu_sc as plsc

B = 256
SEQ = 201
D = 768
M = 100000

_NC = 2
_NS = 16
_ROWS_PER_TILE = B // (_NC * _NS)


def _sc_gather_body(table_hbm, idx_hbm, out_hbm, idx_v, rows_v, sem):
  wid = lax.axis_index("s") * _NC + lax.axis_index("c")
  base = wid * _ROWS_PER_TILE
  pltpu.sync_copy(idx_hbm.at[pl.ds(base, _ROWS_PER_TILE)], idx_v)
  pltpu.async_copy(table_hbm.at[idx_v], rows_v, sem).wait()
  pltpu.sync_copy(rows_v, out_hbm.at[pl.ds(base, _ROWS_PER_TILE)])


def _sc_gather(table, idx):
  gather = pl.kernel(
      _sc_gather_body,
      out_type=jax.ShapeDtypeStruct((B, D), jnp.float32),
      mesh=plsc.VectorSubcoreMesh(core_axis_name="c", subcore_axis_name="s",
                                  num_cores=_NC, num_subcores=_NS),
      scratch_types=[
          pltpu.VMEM((_ROWS_PER_TILE,), jnp.int32),
          pltpu.VMEM((_ROWS_PER_TILE, D), jnp.float32),
          pltpu.SemaphoreType.DMA,
      ],
  )
  return gather(table, idx)


def _compute_body(t_ref, g_ref, wt_ref, bt_ref, wg_ref, bg_ref,
                  ic_ref, ir_ref, tnew_ref, gsend_ref):
  t = t_ref[...]
  g = g_ref[...]
  tt = lax.dot_general(t, wt_ref[...], (((1,), (1,)), ((), ())),
                       preferred_element_type=jnp.float32) + bt_ref[...]
  a = jnp.sum(t * tt, axis=1, keepdims=True)
  b = jnp.sum(g * t, axis=1, keepdims=True)
  m = jnp.maximum(a, b)
  ea = jnp.exp(a - m)
  eb = jnp.exp(b - m)
  s = ea + eb
  tnew_ref[...] = (ea / s) * t + (eb / s) * g

  gt = lax.dot_general(g, wg_ref[...], (((1,), (1,)), ((), ())),
                       preferred_element_type=jnp.float32) + bg_ref[...]
  c = jnp.sum(gt * g, axis=1, keepdims=True)
  m2 = jnp.maximum(c, b)
  ec = jnp.exp(c - m2)
  ed = jnp.exp(b - m2)
  s2 = ec + ed
  graph = (ec / s2) * g + (ed / s2) * t

  eqf = (ic_ref[...] == ir_ref[...]).astype(jnp.float32)
  ki = lax.broadcasted_iota(jnp.int32, (B, B), 0)
  ji = lax.broadcasted_iota(jnp.int32, (B, B), 1)
  upper = (ki > ji).astype(jnp.float32)
  suffix = lax.dot_general(eqf, upper, (((1,), (0,)), ((), ())),
                           preferred_element_type=jnp.float32)
  sel = eqf * (suffix == 0).astype(jnp.float32)
  gsend_ref[...] = lax.dot_general(sel, graph, (((1,), (0,)), ((), ())),
                                   preferred_element_type=jnp.float32)


_compute = pl.pallas_call(
    _compute_body,
    out_shape=(
        jax.ShapeDtypeStruct((B, D), jnp.float32),
        jax.ShapeDtypeStruct((B, D), jnp.float32),
    ),
)


def _settext_body(text2d_any, tnew_ref, out_ref):
  del text2d_any
  out_ref[...] = tnew_ref[...]


_settext = pl.pallas_call(
    _settext_body,
    out_shape=jax.ShapeDtypeStruct((B, SEQ * D), jnp.float32),
    grid=(1,),
    in_specs=[
        pl.BlockSpec(memory_space=pl.ANY),
        pl.BlockSpec((B, D), lambda i: (0, 0)),
    ],
    out_specs=pl.BlockSpec((B, D), lambda i: (0, 0)),
    input_output_aliases={0: 0},
)


def _scatter_body(idx_ref, afu_any, gsend_ref, out_ref):
  del idx_ref, afu_any
  out_ref[...] = gsend_ref[...]


_scatter = pl.pallas_call(
    _scatter_body,
    grid_spec=pltpu.PrefetchScalarGridSpec(
        num_scalar_prefetch=1,
        grid=(B,),
        in_specs=[
            pl.BlockSpec(memory_space=pl.ANY),
            pl.BlockSpec((1, 1, D), lambda i, idx: (i, 0, 0)),
        ],
        out_specs=pl.BlockSpec((1, 1, D), lambda i, idx: (idx[i], 0, 0)),
    ),
    out_shape=jax.ShapeDtypeStruct((M, 1, D), jnp.float32),
    input_output_aliases={1: 0},
)


def kernel(text, all_user_feature, user_neighbor_index, W_text, b_text,
           W_graph, b_graph):
  idx = user_neighbor_index[:, 0].astype(jnp.int32)
  text_ini = text[:, 0, :]

  graph_ini = _sc_gather(all_user_feature, idx)

  text_new, graph_send = _compute(
      text_ini, graph_ini, W_text, b_text.reshape(1, D), W_graph,
      b_graph.reshape(1, D), idx.reshape(B, 1), idx.reshape(1, B))

  text_out = _settext(text.reshape(B, SEQ * D), text_new).reshape(B, SEQ, D)
  afu_out = _scatter(idx, all_user_feature.reshape(M, 1, D),
                     graph_send.reshape(B, 1, D)).reshape(M, D)
  return (text_out, afu_out)


# speedup vs baseline: 2.1094x; 2.1094x over previous
import jax
import jax.numpy as jnp
from jax import lax
from jax.experimental import pallas as pl
from jax.experimental.pallas import tpu as pltpu
from jax.experimental.pallas import tpu_sc as plsc

B = 256
SEQ = 201
D = 768
M = 100000

_NC = 2
_NS = 16
_ROWS_PER_TILE = B // (_NC * _NS)


def _sc_gather_body(table_hbm, idx_hbm, out_hbm, idx_v, rows_v, sem):
  wid = lax.axis_index("s") * _NC + lax.axis_index("c")
  base = wid * _ROWS_PER_TILE
  pltpu.sync_copy(idx_hbm.at[pl.ds(base, _ROWS_PER_TILE)], idx_v)
  pltpu.async_copy(table_hbm.at[idx_v], rows_v, sem).wait()
  pltpu.sync_copy(rows_v, out_hbm.at[pl.ds(base, _ROWS_PER_TILE)])


def _sc_gather(table, idx):
  gather = pl.kernel(
      _sc_gather_body,
      out_type=jax.ShapeDtypeStruct((B, D), jnp.float32),
      mesh=plsc.VectorSubcoreMesh(core_axis_name="c", subcore_axis_name="s",
                                  num_cores=_NC, num_subcores=_NS),
      scratch_types=[
          pltpu.VMEM((_ROWS_PER_TILE,), jnp.int32),
          pltpu.VMEM((_ROWS_PER_TILE, D), jnp.float32),
          pltpu.SemaphoreType.DMA,
      ],
  )
  return gather(table, idx)


def _fused_body(t_ref, g_ref, wt_ref, bt_ref, wg_ref, bg_ref,
                ic_ref, ir_ref, idxs_ref, text_any, afu_any,
                text_out, afu_out, tnew_v, gsend_v, semt, sems):
  del text_any, afu_any
  t = t_ref[...]
  g = g_ref[...]
  tt = lax.dot_general(t, wt_ref[...], (((1,), (1,)), ((), ())),
                       preferred_element_type=jnp.float32) + bt_ref[...]
  a = jnp.sum(t * tt, axis=1, keepdims=True)
  b = jnp.sum(g * t, axis=1, keepdims=True)
  m = jnp.maximum(a, b)
  ea = jnp.exp(a - m)
  eb = jnp.exp(b - m)
  s = ea + eb
  tnew_v[...] = (ea / s) * t + (eb / s) * g
  text_dma = pltpu.make_async_copy(tnew_v, text_out.at[:, pl.ds(0, D)], semt)
  text_dma.start()

  gt = lax.dot_general(g, wg_ref[...], (((1,), (1,)), ((), ())),
                       preferred_element_type=jnp.float32) + bg_ref[...]
  c = jnp.sum(gt * g, axis=1, keepdims=True)
  m2 = jnp.maximum(c, b)
  ec = jnp.exp(c - m2)
  ed = jnp.exp(b - m2)
  s2 = ec + ed
  graph = (ec / s2) * g + (ed / s2) * t

  eqf = (ic_ref[...] == ir_ref[...]).astype(jnp.float32)
  ki = lax.broadcasted_iota(jnp.int32, (B, B), 0)
  ji = lax.broadcasted_iota(jnp.int32, (B, B), 1)
  upper = (ki > ji).astype(jnp.float32)
  suffix = lax.dot_general(eqf, upper, (((1,), (0,)), ((), ())),
                           preferred_element_type=jnp.float32)
  sel = eqf * (suffix == 0).astype(jnp.float32)
  gsend_v[...] = lax.dot_general(sel, graph, (((1,), (0,)), ((), ())),
                                 preferred_element_type=jnp.float32)

  def _start(i, _):
    row = idxs_ref[i]
    pltpu.make_async_copy(gsend_v.at[pl.ds(i, 1)],
                          afu_out.at[pl.ds(row, 1)], sems).start()
    return 0

  lax.fori_loop(0, B, _start, 0)

  def _drain(i, _):
    row = idxs_ref[i]
    pltpu.make_async_copy(gsend_v.at[pl.ds(i, 1)],
                          afu_out.at[pl.ds(row, 1)], sems).wait()
    return 0

  lax.fori_loop(0, B, _drain, 0)
  text_dma.wait()


_fused = pl.pallas_call(
    _fused_body,
    grid=(1,),
    in_specs=[
        pl.BlockSpec((B, D), lambda i: (0, 0)),
        pl.BlockSpec((B, D), lambda i: (0, 0)),
        pl.BlockSpec((D, D), lambda i: (0, 0)),
        pl.BlockSpec((1, D), lambda i: (0, 0)),
        pl.BlockSpec((D, D), lambda i: (0, 0)),
        pl.BlockSpec((1, D), lambda i: (0, 0)),
        pl.BlockSpec((B, 1), lambda i: (0, 0)),
        pl.BlockSpec((1, B), lambda i: (0, 0)),
        pl.BlockSpec(memory_space=pltpu.MemorySpace.SMEM),
        pl.BlockSpec(memory_space=pl.ANY),
        pl.BlockSpec(memory_space=pl.ANY),
    ],
    out_specs=(
        pl.BlockSpec(memory_space=pl.ANY),
        pl.BlockSpec(memory_space=pl.ANY),
    ),
    out_shape=(
        jax.ShapeDtypeStruct((B, SEQ * D), jnp.float32),
        jax.ShapeDtypeStruct((M, D), jnp.float32),
    ),
    scratch_shapes=[
        pltpu.VMEM((B, D), jnp.float32),
        pltpu.VMEM((B, D), jnp.float32),
        pltpu.SemaphoreType.DMA,
        pltpu.SemaphoreType.DMA,
    ],
    input_output_aliases={9: 0, 10: 1},
)


def kernel(text, all_user_feature, user_neighbor_index, W_text, b_text,
           W_graph, b_graph):
  idx = user_neighbor_index[:, 0].astype(jnp.int32)
  text_ini = text[:, 0, :]

  graph_ini = _sc_gather(all_user_feature, idx)

  text2d, afu_out = _fused(
      text_ini, graph_ini, W_text, b_text.reshape(1, D), W_graph,
      b_graph.reshape(1, D), idx.reshape(B, 1), idx.reshape(1, B), idx,
      text.reshape(B, SEQ * D), all_user_feature)
  return (text2d.reshape(B, SEQ, D), afu_out)
